# baseline (speedup 1.0000x reference)
import functools

import jax
import jax.numpy as jnp
from jax import lax
from jax.experimental import pallas as pl
from jax.experimental.pallas import tpu as pltpu
from jax.experimental.pallas import tpu_sc as plsc

_VOCAB = 1000000
_D = 64
_B = 4096
_L = 200
_TOTAL = _B * _L
_NC = 2
_NS = 16
_NW = _NC * _NS
_PER_W = _TOTAL // _NW
_CHUNK = 128
_NCHUNK = _PER_W // _CHUNK
_NBUF = 4
_NGROUP = _NCHUNK // _NBUF
_LANES = 16
_SCALE = 8.0


def _embed_body(table_hbm, x_hbm, out_hbm, idx_v, *rest):
    gbufs = rest[0:_NBUF]
    obufs = rest[_NBUF:2 * _NBUF]
    gsems = rest[2 * _NBUF:3 * _NBUF]
    osems = rest[3 * _NBUF:4 * _NBUF]

    wid = lax.axis_index("s") * _NC + lax.axis_index("c")
    out_base = wid * _PER_W

    pltpu.sync_copy(x_hbm.at[pl.ds(wid * _NCHUNK, _NCHUNK)], idx_v)

    def gather_copy(c, b):
        return pltpu.make_async_copy(
            table_hbm.at[idx_v.at[c]], gbufs[b], gsems[b])

    def out_copy(c, b):
        return pltpu.make_async_copy(
            obufs[b], out_hbm.at[pl.ds(out_base + c * _CHUNK, _CHUNK)],
            osems[b])

    for b in range(_NBUF):
        gather_copy(b, b).start()

    def group(g, carry):
        for b in range(_NBUF):
            c = g * _NBUF + b
            gather_copy(c, b).wait()

            @pl.when(g > 0)
            def _():
                out_copy(c - _NBUF, b).wait()

            gbuf = gbufs[b]
            obuf = obufs[b]

            def row(i, acc, gbuf=gbuf, obuf=obuf):
                for k in range(_D // _LANES):
                    sl = pl.ds(k * _LANES, _LANES)
                    obuf[i, sl] = gbuf[i, sl] * _SCALE
                return acc

            lax.fori_loop(0, _CHUNK, row, 0, unroll=8)

            out_copy(c, b).start()

            @pl.when(g + 1 < _NGROUP)
            def _():
                gather_copy(c + _NBUF, b).start()
        return carry

    lax.fori_loop(0, _NGROUP, group, 0)

    for b in range(_NBUF):
        out_copy((_NGROUP - 1) * _NBUF + b, b).wait()


_embed_call = functools.partial(
    pl.kernel,
    out_type=jax.ShapeDtypeStruct((_TOTAL, _D), jnp.float32),
    mesh=plsc.VectorSubcoreMesh(core_axis_name="c", subcore_axis_name="s"),
    compiler_params=pltpu.CompilerParams(use_tc_tiling_on_sc=False),
    scratch_types=(
        [pltpu.VMEM((_NCHUNK, _CHUNK), jnp.int32)]
        + [pltpu.VMEM((_CHUNK, _D), jnp.float32) for _ in range(2 * _NBUF)]
        + [pltpu.SemaphoreType.DMA for _ in range(2 * _NBUF)]
    ),
)(_embed_body)


@jax.jit
def kernel(embedding_table, x):
    xf = x.reshape(_NW * _NCHUNK, _CHUNK)
    out = _embed_call(embedding_table, xf)
    return out.reshape(_B, _L, _D)
